# Initial kernel scaffold; baseline (speedup 1.0000x reference)
import jax
import jax.numpy as jnp
from jax.experimental import pallas as pl


def kernel(feat, edge_index, eps):
    raise NotImplementedError("write your pallas kernel here")


# speedup vs baseline: 5.0821x; 5.0821x over previous
import jax
import jax.numpy as jnp
from jax import lax
from jax.experimental import pallas as pl
from jax.experimental.pallas import tpu as pltpu
from jax.experimental.pallas import tpu_sc as plsc

N_NODES = 10000
N_PAD = 10240
N_EDGES = 320000
D_FEAT = 128

NC = 2
NS = 16
EPC = N_EDGES // NC
EPT = EPC // NS
RPT = N_PAD // NS
CH = 80
NCHUNK = EPT // CH
RB = 64
NRB = RPT // RB


def _gin_body(feat_hbm, src_hbm, dst_hbm, out_hbm,
              rows_v, abuf, stage_src, stage_dst, acc_sh, sem):
    c = lax.axis_index("c")
    s = lax.axis_index("s")
    r0 = s * RPT
    e0 = (c * NS + s) * EPT

    for k in range(NRB):
        rr = r0 + k * RB
        pltpu.sync_copy(feat_hbm.at[pl.ds(rr, RB), :], abuf)
        pltpu.sync_copy(abuf, acc_sh.at[pl.ds(rr, RB)])

    plsc.subcore_barrier()

    def chunk_body(i, carry):
        off = e0 + i * CH
        pltpu.sync_copy(src_hbm.at[pl.ds(off, CH)], stage_src)
        pltpu.sync_copy(dst_hbm.at[pl.ds(off, CH)], stage_dst)
        pltpu.async_copy(feat_hbm.at[stage_src], rows_v, sem).wait()
        pltpu.sync_copy(rows_v, acc_sh.at[stage_dst], add=True)
        return carry

    lax.fori_loop(0, NCHUNK, chunk_body, 0)

    plsc.subcore_barrier()

    for k in range(NRB):
        rr = r0 + k * RB
        pltpu.sync_copy(acc_sh.at[pl.ds(rr, RB)], abuf)
        pltpu.sync_copy(abuf, out_hbm.at[c, pl.ds(rr, RB), :])


@jax.jit
def _gin_sc(feat_p, src32, dst32):
    mesh = plsc.VectorSubcoreMesh(core_axis_name="c", subcore_axis_name="s")
    run = pl.kernel(
        _gin_body,
        out_type=jax.ShapeDtypeStruct((NC, N_PAD, D_FEAT), jnp.float32),
        mesh=mesh,
        scratch_types=[
            pltpu.VMEM((CH, D_FEAT), jnp.float32),
            pltpu.VMEM((RB, D_FEAT), jnp.float32),
            pltpu.VMEM((CH,), jnp.int32),
            pltpu.VMEM((CH,), jnp.int32),
            pltpu.VMEM_SHARED((N_PAD, D_FEAT), jnp.float32),
            pltpu.SemaphoreType.DMA,
        ],
    )
    return run(feat_p, src32, dst32)


def _combine_body(em1_ref, f_ref, p_ref, o_ref):
    em1 = em1_ref[0]
    o_ref[...] = jax.nn.sigmoid(p_ref[0] + p_ref[1] + em1 * f_ref[...])


TB = 256


@jax.jit
def _combine_tc(epsm1, feat_p, parts):
    return pl.pallas_call(
        _combine_body,
        out_shape=jax.ShapeDtypeStruct((N_PAD, D_FEAT), jnp.float32),
        grid=(N_PAD // TB,),
        in_specs=[
            pl.BlockSpec(memory_space=pltpu.SMEM),
            pl.BlockSpec((TB, D_FEAT), lambda i: (i, 0)),
            pl.BlockSpec((NC, TB, D_FEAT), lambda i: (0, i, 0)),
        ],
        out_specs=pl.BlockSpec((TB, D_FEAT), lambda i: (i, 0)),
    )(epsm1, feat_p, parts)


def kernel(feat, edge_index, eps):
    feat_p = jnp.pad(feat, ((0, N_PAD - N_NODES), (0, 0)))
    edge32 = edge_index.astype(jnp.int32)
    parts = _gin_sc(feat_p, edge32[0], edge32[1])
    epsm1 = (eps - 1.0).astype(jnp.float32)
    return _combine_tc(epsm1, feat_p, parts)[:N_NODES]


# speedup vs baseline: 7.7391x; 1.5228x over previous
import jax
import jax.numpy as jnp
from jax import lax
from jax.experimental import pallas as pl
from jax.experimental.pallas import tpu as pltpu
from jax.experimental.pallas import tpu_sc as plsc

N_NODES = 10000
N_PAD = 10240
N_EDGES = 320000
D_FEAT = 128

NC = 2
NS = 16
EPT = 10240
E_PAD = NC * NS * EPT
RPT = N_PAD // NS
CH = 80
NCHUNK = EPT // CH
NJ = NCHUNK // 2


def _gin_body_impl(feat_hbm, src_hbm, dst_hbm, out_hbm,
                   rows_a, rows_b, sa_src, sa_dst, sb_src, sb_dst,
                   acc_sh, sem_a, sem_b):
    c = lax.axis_index("c")
    s = lax.axis_index("s")
    r0 = s * RPT
    e0 = (c * NS + s) * EPT

    pltpu.sync_copy(feat_hbm.at[pl.ds(r0, RPT), :], acc_sh.at[pl.ds(r0, RPT)])

    plsc.subcore_barrier()

    def stage_and_fire(i, src_st, dst_st, rows, sem):
        off = e0 + i * CH
        pltpu.sync_copy(src_hbm.at[pl.ds(off, CH)], src_st)
        pltpu.sync_copy(dst_hbm.at[pl.ds(off, CH)], dst_st)
        return pltpu.async_copy(feat_hbm.at[src_st], rows, sem)

    stage_and_fire(0, sa_src, sa_dst, rows_a, sem_a)

    def pipe_body(j, carry):
        stage_and_fire(2 * j + 1, sb_src, sb_dst, rows_b, sem_b)
        pltpu.make_async_copy(feat_hbm.at[sa_src], rows_a, sem_a).wait()
        pltpu.sync_copy(rows_a, acc_sh.at[sa_dst], add=True)

        @pl.when(j < NJ - 1)
        def _():
            stage_and_fire(2 * j + 2, sa_src, sa_dst, rows_a, sem_a)

        pltpu.make_async_copy(feat_hbm.at[sb_src], rows_b, sem_b).wait()
        pltpu.sync_copy(rows_b, acc_sh.at[sb_dst], add=True)
        return carry

    lax.fori_loop(0, NJ, pipe_body, 0)

    plsc.subcore_barrier()

    pltpu.sync_copy(acc_sh.at[pl.ds(r0, RPT)], out_hbm.at[c, pl.ds(r0, RPT), :])


@jax.jit
def _gin_sc(feat_p, src32, dst32):
    mesh = plsc.VectorSubcoreMesh(core_axis_name="c", subcore_axis_name="s")
    run = pl.kernel(
        _gin_body_impl,
        out_type=jax.ShapeDtypeStruct((NC, N_PAD, D_FEAT), jnp.float32),
        mesh=mesh,
        scratch_types=[
            pltpu.VMEM((CH, D_FEAT), jnp.float32),
            pltpu.VMEM((CH, D_FEAT), jnp.float32),
            pltpu.VMEM((CH,), jnp.int32),
            pltpu.VMEM((CH,), jnp.int32),
            pltpu.VMEM((CH,), jnp.int32),
            pltpu.VMEM((CH,), jnp.int32),
            pltpu.VMEM_SHARED((N_PAD, D_FEAT), jnp.float32),
            pltpu.SemaphoreType.DMA,
            pltpu.SemaphoreType.DMA,
        ],
    )
    return run(feat_p, src32, dst32)


def _combine_body(em1_ref, f_ref, p_ref, o_ref):
    em1 = em1_ref[0]
    o_ref[...] = jax.nn.sigmoid(p_ref[0] + p_ref[1] + em1 * f_ref[...])


TB = 256


@jax.jit
def _combine_tc(epsm1, feat_p, parts):
    return pl.pallas_call(
        _combine_body,
        out_shape=jax.ShapeDtypeStruct((N_PAD, D_FEAT), jnp.float32),
        grid=(N_PAD // TB,),
        in_specs=[
            pl.BlockSpec(memory_space=pltpu.SMEM),
            pl.BlockSpec((TB, D_FEAT), lambda i: (i, 0)),
            pl.BlockSpec((NC, TB, D_FEAT), lambda i: (0, i, 0)),
        ],
        out_specs=pl.BlockSpec((TB, D_FEAT), lambda i: (i, 0)),
    )(epsm1, feat_p, parts)


def kernel(feat, edge_index, eps):
    feat_p = jnp.pad(feat, ((0, N_PAD - N_NODES), (0, 0)))
    edge32 = edge_index.astype(jnp.int32)
    pad_idx = N_NODES + (jnp.arange(E_PAD - N_EDGES, dtype=jnp.int32)
                         % (N_PAD - N_NODES))
    src32 = jnp.concatenate([edge32[0], pad_idx])
    dst32 = jnp.concatenate([edge32[1], pad_idx])
    parts = _gin_sc(feat_p, src32, dst32)
    epsm1 = (eps - 1.0).astype(jnp.float32)
    return _combine_tc(epsm1, feat_p, parts)[:N_NODES]


# speedup vs baseline: 9.4569x; 1.2220x over previous
import jax
import jax.numpy as jnp
from jax import lax
from jax.experimental import pallas as pl
from jax.experimental.pallas import tpu as pltpu
from jax.experimental.pallas import tpu_sc as plsc

N_NODES = 10000
N_PAD = 10240
N_EDGES = 320000
D_FEAT = 128

NC = 2
NS = 16
EPT = 10240
E_PAD = NC * NS * EPT
RPT = N_PAD // NS
CH = 64
NCHUNK = EPT // CH
SB = 40
NSB = NCHUNK // SB


def _gin_body(feat_hbm, src_hbm, dst_hbm, out_hbm,
              rows_a, rows_b, src_idx, dst_idx,
              acc_sh, sem_a, sem_b):
    c = lax.axis_index("c")
    s = lax.axis_index("s")
    r0 = s * RPT
    c0 = (c * NS + s) * NCHUNK

    pltpu.sync_copy(feat_hbm.at[pl.ds(r0, RPT), :], acc_sh.at[pl.ds(r0, RPT)])

    plsc.subcore_barrier()

    for b in range(NSB):
        cb = c0 + b * SB
        pltpu.sync_copy(src_hbm.at[pl.ds(cb, SB), :], src_idx)
        pltpu.sync_copy(dst_hbm.at[pl.ds(cb, SB), :], dst_idx)

        pltpu.async_copy(feat_hbm.at[src_idx.at[0]], rows_a, sem_a)

        def pipe_body(j, carry):
            pltpu.async_copy(feat_hbm.at[src_idx.at[2 * j + 1]], rows_b,
                             sem_b)
            pltpu.make_async_copy(feat_hbm.at[src_idx.at[2 * j]], rows_a,
                                  sem_a).wait()
            pltpu.sync_copy(rows_a, acc_sh.at[dst_idx.at[2 * j]], add=True)

            @pl.when(j < SB // 2 - 1)
            def _():
                pltpu.async_copy(feat_hbm.at[src_idx.at[2 * j + 2]], rows_a,
                                 sem_a)

            pltpu.make_async_copy(feat_hbm.at[src_idx.at[2 * j + 1]], rows_b,
                                  sem_b).wait()
            pltpu.sync_copy(rows_b, acc_sh.at[dst_idx.at[2 * j + 1]],
                            add=True)
            return carry

        lax.fori_loop(0, SB // 2, pipe_body, 0)

    plsc.subcore_barrier()

    pltpu.sync_copy(acc_sh.at[pl.ds(r0, RPT)], out_hbm.at[c, pl.ds(r0, RPT), :])


@jax.jit
def _gin_sc(feat_p, src2d, dst2d):
    mesh = plsc.VectorSubcoreMesh(core_axis_name="c", subcore_axis_name="s")
    run = pl.kernel(
        _gin_body,
        out_type=jax.ShapeDtypeStruct((NC, N_PAD, D_FEAT), jnp.float32),
        mesh=mesh,
        scratch_types=[
            pltpu.VMEM((CH, D_FEAT), jnp.float32),
            pltpu.VMEM((CH, D_FEAT), jnp.float32),
            pltpu.VMEM((SB, CH), jnp.int32),
            pltpu.VMEM((SB, CH), jnp.int32),
            pltpu.VMEM_SHARED((N_PAD, D_FEAT), jnp.float32),
            pltpu.SemaphoreType.DMA,
            pltpu.SemaphoreType.DMA,
        ],
    )
    return run(feat_p, src2d, dst2d)


def _combine_body(em1_ref, f_ref, p_ref, o_ref):
    em1 = em1_ref[0]
    o_ref[...] = jax.nn.sigmoid(p_ref[0] + p_ref[1] + em1 * f_ref[...])


TB = 256


@jax.jit
def _combine_tc(epsm1, feat_p, parts):
    return pl.pallas_call(
        _combine_body,
        out_shape=jax.ShapeDtypeStruct((N_PAD, D_FEAT), jnp.float32),
        grid=(N_PAD // TB,),
        in_specs=[
            pl.BlockSpec(memory_space=pltpu.SMEM),
            pl.BlockSpec((TB, D_FEAT), lambda i: (i, 0)),
            pl.BlockSpec((NC, TB, D_FEAT), lambda i: (0, i, 0)),
        ],
        out_specs=pl.BlockSpec((TB, D_FEAT), lambda i: (i, 0)),
    )(epsm1, feat_p, parts)


def kernel(feat, edge_index, eps):
    feat_p = jnp.pad(feat, ((0, N_PAD - N_NODES), (0, 0)))
    edge32 = edge_index.astype(jnp.int32)
    pad_idx = N_NODES + (jnp.arange(E_PAD - N_EDGES, dtype=jnp.int32)
                         % (N_PAD - N_NODES))
    src2d = jnp.concatenate([edge32[0], pad_idx]).reshape(E_PAD // CH, CH)
    dst2d = jnp.concatenate([edge32[1], pad_idx]).reshape(E_PAD // CH, CH)
    parts = _gin_sc(feat_p, src2d, dst2d)
    epsm1 = (eps - 1.0).astype(jnp.float32)
    return _combine_tc(epsm1, feat_p, parts)[:N_NODES]


# speedup vs baseline: 10.2844x; 1.0875x over previous
import jax
import jax.numpy as jnp
from jax import lax
from jax.experimental import pallas as pl
from jax.experimental.pallas import tpu as pltpu
from jax.experimental.pallas import tpu_sc as plsc

N_NODES = 10000
N_PAD = 10240
N_EDGES = 320000
D_FEAT = 128

NC = 2
NS = 16
EPT = 10240
E_PAD = NC * NS * EPT
RPT = N_PAD // NS
R_LAST = N_NODES - (NS - 1) * RPT
CH = 64
NCHUNK = EPT // CH
SB = 40
NSB = NCHUNK // SB


def _gin_body(feat_hbm, src_hbm, dst_hbm, out_hbm,
              rows_a, rows_b, src_idx, dst_idx,
              acc_sh, sem_a, sem_b):
    c = lax.axis_index("c")
    s = lax.axis_index("s")
    r0 = s * RPT
    c0 = (c * NS + s) * NCHUNK

    @pl.when(s < NS - 1)
    def _():
        pltpu.sync_copy(feat_hbm.at[pl.ds(r0, RPT), :],
                        acc_sh.at[pl.ds(r0, RPT)])

    @pl.when(s == NS - 1)
    def _():
        pltpu.sync_copy(feat_hbm.at[pl.ds(r0, R_LAST), :],
                        acc_sh.at[pl.ds(r0, R_LAST)])

    plsc.subcore_barrier()

    for b in range(NSB):
        cb = c0 + b * SB
        pltpu.sync_copy(src_hbm.at[pl.ds(cb, SB), :], src_idx)
        pltpu.sync_copy(dst_hbm.at[pl.ds(cb, SB), :], dst_idx)

        pltpu.async_copy(feat_hbm.at[src_idx.at[0]], rows_a, sem_a)

        def pipe_body(j, carry):
            pltpu.async_copy(feat_hbm.at[src_idx.at[2 * j + 1]], rows_b,
                             sem_b)
            pltpu.make_async_copy(feat_hbm.at[src_idx.at[2 * j]], rows_a,
                                  sem_a).wait()
            pltpu.sync_copy(rows_a, acc_sh.at[dst_idx.at[2 * j]], add=True)

            @pl.when(j < SB // 2 - 1)
            def _():
                pltpu.async_copy(feat_hbm.at[src_idx.at[2 * j + 2]], rows_a,
                                 sem_a)

            pltpu.make_async_copy(feat_hbm.at[src_idx.at[2 * j + 1]], rows_b,
                                  sem_b).wait()
            pltpu.sync_copy(rows_b, acc_sh.at[dst_idx.at[2 * j + 1]],
                            add=True)
            return carry

        lax.fori_loop(0, SB // 2, pipe_body, 0)

    plsc.subcore_barrier()

    @pl.when(s < NS - 1)
    def _():
        pltpu.sync_copy(acc_sh.at[pl.ds(r0, RPT)],
                        out_hbm.at[c, pl.ds(r0, RPT), :])

    @pl.when(s == NS - 1)
    def _():
        pltpu.sync_copy(acc_sh.at[pl.ds(r0, R_LAST)],
                        out_hbm.at[c, pl.ds(r0, R_LAST), :])


@jax.jit
def _gin_sc(feat_p, src2d, dst2d):
    mesh = plsc.VectorSubcoreMesh(core_axis_name="c", subcore_axis_name="s")
    run = pl.kernel(
        _gin_body,
        out_type=jax.ShapeDtypeStruct((NC, N_NODES, D_FEAT), jnp.float32),
        mesh=mesh,
        scratch_types=[
            pltpu.VMEM((CH, D_FEAT), jnp.float32),
            pltpu.VMEM((CH, D_FEAT), jnp.float32),
            pltpu.VMEM((SB, CH), jnp.int32),
            pltpu.VMEM((SB, CH), jnp.int32),
            pltpu.VMEM_SHARED((N_PAD, D_FEAT), jnp.float32),
            pltpu.SemaphoreType.DMA,
            pltpu.SemaphoreType.DMA,
        ],
    )
    return run(feat_p, src2d, dst2d)


def _combine_body(em1_ref, f_ref, p_ref, o_ref):
    em1 = em1_ref[0]
    o_ref[...] = jax.nn.sigmoid(p_ref[0] + p_ref[1] + em1 * f_ref[...])


TB = 400


@jax.jit
def _combine_tc(epsm1, feat_p, parts):
    return pl.pallas_call(
        _combine_body,
        out_shape=jax.ShapeDtypeStruct((N_NODES, D_FEAT), jnp.float32),
        grid=(N_NODES // TB,),
        in_specs=[
            pl.BlockSpec(memory_space=pltpu.SMEM),
            pl.BlockSpec((TB, D_FEAT), lambda i: (i, 0)),
            pl.BlockSpec((NC, TB, D_FEAT), lambda i: (0, i, 0)),
        ],
        out_specs=pl.BlockSpec((TB, D_FEAT), lambda i: (i, 0)),
    )(epsm1, feat_p, parts)


def kernel(feat, edge_index, eps):
    edge32 = edge_index.astype(jnp.int32)
    n_extra = E_PAD - N_EDGES
    pad_src = jnp.arange(n_extra, dtype=jnp.int32) % N_NODES
    pad_dst = N_NODES + (jnp.arange(n_extra, dtype=jnp.int32)
                         % (N_PAD - N_NODES))
    src2d = jnp.concatenate([edge32[0], pad_src]).reshape(E_PAD // CH, CH)
    dst2d = jnp.concatenate([edge32[1], pad_dst]).reshape(E_PAD // CH, CH)
    parts = _gin_sc(feat, src2d, dst2d)
    epsm1 = (eps - 1.0).astype(jnp.float32)
    return _combine_tc(epsm1, feat, parts)


# speedup vs baseline: 10.6703x; 1.0375x over previous
import jax
import jax.numpy as jnp
from jax import lax
from jax.experimental import pallas as pl
from jax.experimental.pallas import tpu as pltpu
from jax.experimental.pallas import tpu_sc as plsc

N_NODES = 10000
N_PAD = 10240
N_EDGES = 320000
D_FEAT = 128

NC = 2
NS = 16
EPT = 10240
E_PAD = NC * NS * EPT
RPT = N_PAD // NS
R_LAST = N_NODES - (NS - 1) * RPT
CH = 32
NCHUNK = EPT // CH
SB = 40
NSB = NCHUNK // SB
NQ = SB // 4


def _gin_body(feat_hbm, src_hbm, dst_hbm, out_hbm,
              rows_a, rows_b, rows_c, rows_d, src_idx, dst_idx,
              acc_sh, sem_a, sem_b, sem_c, sem_d):
    c = lax.axis_index("c")
    s = lax.axis_index("s")
    r0 = s * RPT
    c0 = (c * NS + s) * NCHUNK

    @pl.when(s < NS - 1)
    def _():
        pltpu.sync_copy(feat_hbm.at[pl.ds(r0, RPT), :],
                        acc_sh.at[pl.ds(r0, RPT)])

    @pl.when(s == NS - 1)
    def _():
        pltpu.sync_copy(feat_hbm.at[pl.ds(r0, R_LAST), :],
                        acc_sh.at[pl.ds(r0, R_LAST)])

    plsc.subcore_barrier()

    bufs = ((rows_a, sem_a), (rows_b, sem_b), (rows_c, sem_c),
            (rows_d, sem_d))
    for b in range(NSB):
        cb = c0 + b * SB
        pltpu.sync_copy(src_hbm.at[pl.ds(cb, SB), :], src_idx)
        pltpu.sync_copy(dst_hbm.at[pl.ds(cb, SB), :], dst_idx)

        for q, (rows, sem) in enumerate(bufs):
            pltpu.async_copy(feat_hbm.at[src_idx.at[q]], rows, sem)

        def pipe_body(j, carry):
            for q, (rows, sem) in enumerate(bufs):
                i = 4 * j + q
                pltpu.make_async_copy(feat_hbm.at[src_idx.at[i]], rows,
                                      sem).wait()
                pltpu.sync_copy(rows, acc_sh.at[dst_idx.at[i]], add=True)

                @pl.when(j < NQ - 1)
                def _():
                    pltpu.async_copy(feat_hbm.at[src_idx.at[i + 4]], rows,
                                     sem)
            return carry

        lax.fori_loop(0, NQ, pipe_body, 0)

    plsc.subcore_barrier()

    @pl.when(s < NS - 1)
    def _():
        pltpu.sync_copy(acc_sh.at[pl.ds(r0, RPT)],
                        out_hbm.at[c, pl.ds(r0, RPT), :])

    @pl.when(s == NS - 1)
    def _():
        pltpu.sync_copy(acc_sh.at[pl.ds(r0, R_LAST)],
                        out_hbm.at[c, pl.ds(r0, R_LAST), :])


@jax.jit
def _gin_sc(feat_p, src2d, dst2d):
    mesh = plsc.VectorSubcoreMesh(core_axis_name="c", subcore_axis_name="s")
    run = pl.kernel(
        _gin_body,
        out_type=jax.ShapeDtypeStruct((NC, N_NODES, D_FEAT), jnp.float32),
        mesh=mesh,
        scratch_types=[
            pltpu.VMEM((CH, D_FEAT), jnp.float32),
            pltpu.VMEM((CH, D_FEAT), jnp.float32),
            pltpu.VMEM((CH, D_FEAT), jnp.float32),
            pltpu.VMEM((CH, D_FEAT), jnp.float32),
            pltpu.VMEM((SB, CH), jnp.int32),
            pltpu.VMEM((SB, CH), jnp.int32),
            pltpu.VMEM_SHARED((N_PAD, D_FEAT), jnp.float32),
            pltpu.SemaphoreType.DMA,
            pltpu.SemaphoreType.DMA,
            pltpu.SemaphoreType.DMA,
            pltpu.SemaphoreType.DMA,
        ],
    )
    return run(feat_p, src2d, dst2d)


def _combine_body(em1_ref, f_ref, p_ref, o_ref):
    em1 = em1_ref[0]
    o_ref[...] = jax.nn.sigmoid(p_ref[0] + p_ref[1] + em1 * f_ref[...])


TB = 400


@jax.jit
def _combine_tc(epsm1, feat_p, parts):
    return pl.pallas_call(
        _combine_body,
        out_shape=jax.ShapeDtypeStruct((N_NODES, D_FEAT), jnp.float32),
        grid=(N_NODES // TB,),
        in_specs=[
            pl.BlockSpec(memory_space=pltpu.SMEM),
            pl.BlockSpec((TB, D_FEAT), lambda i: (i, 0)),
            pl.BlockSpec((NC, TB, D_FEAT), lambda i: (0, i, 0)),
        ],
        out_specs=pl.BlockSpec((TB, D_FEAT), lambda i: (i, 0)),
    )(epsm1, feat_p, parts)


def kernel(feat, edge_index, eps):
    edge32 = edge_index.astype(jnp.int32)
    n_extra = E_PAD - N_EDGES
    pad_src = jnp.arange(n_extra, dtype=jnp.int32) % N_NODES
    pad_dst = N_NODES + (jnp.arange(n_extra, dtype=jnp.int32)
                         % (N_PAD - N_NODES))
    src2d = jnp.concatenate([edge32[0], pad_src]).reshape(E_PAD // CH, CH)
    dst2d = jnp.concatenate([edge32[1], pad_dst]).reshape(E_PAD // CH, CH)
    parts = _gin_sc(feat, src2d, dst2d)
    epsm1 = (eps - 1.0).astype(jnp.float32)
    return _combine_tc(epsm1, feat, parts)


# speedup vs baseline: 10.9683x; 1.0279x over previous
import jax
import jax.numpy as jnp
from jax import lax
from jax.experimental import pallas as pl
from jax.experimental.pallas import tpu as pltpu
from jax.experimental.pallas import tpu_sc as plsc

N_NODES = 10000
N_PAD = 10240
N_EDGES = 320000
D_FEAT = 128

NC = 2
NS = 16
EPT = 10240
E_PAD = NC * NS * EPT
RPT = N_PAD // NS
R_LAST = N_NODES - (NS - 1) * RPT
CH = 40
NCHUNK = EPT // CH
SB = 32
NSB = NCHUNK // SB
NQ = SB // 4


def _gin_body(feat_hbm, src_hbm, dst_hbm, out_hbm,
              rows_a, rows_b, rows_c, rows_d, src_idx, dst_idx,
              acc_sh, sem_a, sem_b, sem_c, sem_d):
    c = lax.axis_index("c")
    s = lax.axis_index("s")
    r0 = s * RPT
    c0 = (c * NS + s) * NCHUNK

    @pl.when(s < NS - 1)
    def _():
        pltpu.sync_copy(feat_hbm.at[pl.ds(r0, RPT), :],
                        acc_sh.at[pl.ds(r0, RPT)])

    @pl.when(s == NS - 1)
    def _():
        pltpu.sync_copy(feat_hbm.at[pl.ds(r0, R_LAST), :],
                        acc_sh.at[pl.ds(r0, R_LAST)])

    plsc.subcore_barrier()

    bufs = ((rows_a, sem_a), (rows_b, sem_b), (rows_c, sem_c),
            (rows_d, sem_d))
    for b in range(NSB):
        cb = c0 + b * SB
        pltpu.sync_copy(src_hbm.at[pl.ds(cb, SB), :], src_idx)
        pltpu.sync_copy(dst_hbm.at[pl.ds(cb, SB), :], dst_idx)

        for q, (rows, sem) in enumerate(bufs):
            pltpu.async_copy(feat_hbm.at[src_idx.at[q]], rows, sem)

        def pipe_body(j, carry):
            for q, (rows, sem) in enumerate(bufs):
                i = 4 * j + q
                pltpu.make_async_copy(feat_hbm.at[src_idx.at[i]], rows,
                                      sem).wait()
                pltpu.sync_copy(rows, acc_sh.at[dst_idx.at[i]], add=True)

                @pl.when(j < NQ - 1)
                def _():
                    pltpu.async_copy(feat_hbm.at[src_idx.at[i + 4]], rows,
                                     sem)
            return carry

        lax.fori_loop(0, NQ, pipe_body, 0)

    plsc.subcore_barrier()

    @pl.when(s < NS - 1)
    def _():
        pltpu.sync_copy(acc_sh.at[pl.ds(r0, RPT)],
                        out_hbm.at[c, pl.ds(r0, RPT), :])

    @pl.when(s == NS - 1)
    def _():
        pltpu.sync_copy(acc_sh.at[pl.ds(r0, R_LAST)],
                        out_hbm.at[c, pl.ds(r0, R_LAST), :])


@jax.jit
def _gin_sc(feat_p, src2d, dst2d):
    mesh = plsc.VectorSubcoreMesh(core_axis_name="c", subcore_axis_name="s")
    run = pl.kernel(
        _gin_body,
        out_type=jax.ShapeDtypeStruct((NC, N_NODES, D_FEAT), jnp.float32),
        mesh=mesh,
        scratch_types=[
            pltpu.VMEM((CH, D_FEAT), jnp.float32),
            pltpu.VMEM((CH, D_FEAT), jnp.float32),
            pltpu.VMEM((CH, D_FEAT), jnp.float32),
            pltpu.VMEM((CH, D_FEAT), jnp.float32),
            pltpu.VMEM((SB, CH), jnp.int32),
            pltpu.VMEM((SB, CH), jnp.int32),
            pltpu.VMEM_SHARED((N_PAD, D_FEAT), jnp.float32),
            pltpu.SemaphoreType.DMA,
            pltpu.SemaphoreType.DMA,
            pltpu.SemaphoreType.DMA,
            pltpu.SemaphoreType.DMA,
        ],
    )
    return run(feat_p, src2d, dst2d)


def _combine_body(em1_ref, f_ref, p_ref, o_ref):
    em1 = em1_ref[0]
    o_ref[...] = jax.nn.sigmoid(p_ref[0] + p_ref[1] + em1 * f_ref[...])


TB = 400


@jax.jit
def _combine_tc(epsm1, feat_p, parts):
    return pl.pallas_call(
        _combine_body,
        out_shape=jax.ShapeDtypeStruct((N_NODES, D_FEAT), jnp.float32),
        grid=(N_NODES // TB,),
        in_specs=[
            pl.BlockSpec(memory_space=pltpu.SMEM),
            pl.BlockSpec((TB, D_FEAT), lambda i: (i, 0)),
            pl.BlockSpec((NC, TB, D_FEAT), lambda i: (0, i, 0)),
        ],
        out_specs=pl.BlockSpec((TB, D_FEAT), lambda i: (i, 0)),
    )(epsm1, feat_p, parts)


def kernel(feat, edge_index, eps):
    edge32 = edge_index.astype(jnp.int32)
    n_extra = E_PAD - N_EDGES
    pad_src = jnp.arange(n_extra, dtype=jnp.int32) % N_NODES
    pad_dst = N_NODES + (jnp.arange(n_extra, dtype=jnp.int32)
                         % (N_PAD - N_NODES))
    src2d = jnp.concatenate([edge32[0], pad_src]).reshape(E_PAD // CH, CH)
    dst2d = jnp.concatenate([edge32[1], pad_dst]).reshape(E_PAD // CH, CH)
    parts = _gin_sc(feat, src2d, dst2d)
    epsm1 = (eps - 1.0).astype(jnp.float32)
    return _combine_tc(epsm1, feat, parts)
